# baseline (speedup 1.0000x reference)
import jax
import jax.numpy as jnp
from jax import lax
from jax.experimental import pallas as pl
from jax.experimental.pallas import tpu as pltpu
from jax.experimental.pallas import tpu_sc as plsc

B = 16384
CLASS_NUM = 2
D = 1024
LANES = 16
NC = 2
NS = 16
NW = NC * NS

BS = 2048
BT = B - BS
RS = BS // NW
CHUNK = 32
NCHUNK = RS // CHUNK
NJ = D // LANES
NGRP = 4
GW = NJ // NGRP


def _sc_body(feat_hbm, cen_hbm, a0p_hbm, pdp_hbm, cen_v, cd_v, fb0, fb1,
             a0p_v, pdp_v, sem0, sem1):
    wid = lax.axis_index("s") * NC + lax.axis_index("c")
    base = wid * RS

    pltpu.sync_copy(cen_hbm, cen_v)
    for j in range(NJ):
        c0 = cen_v[pl.ds(j * LANES, LANES)]
        c1 = cen_v[pl.ds(D + j * LANES, LANES)]
        cd_v[pl.ds(j * LANES, LANES)] = c0 - c1

    def process(g, fb):
        for jg in range(NGRP):
            cvals = tuple(
                cen_v[pl.ds((jg * GW + m) * LANES, LANES)] for m in range(GW)
            ) + tuple(
                cd_v[pl.ds((jg * GW + m) * LANES, LANES)] for m in range(GW)
            )

            def row_body(r, carry, _jg=jg):
                rr = g * CHUNK + r
                a0 = jnp.zeros((LANES,), jnp.float32)
                pd = jnp.zeros((LANES,), jnp.float32)
                for m in range(GW):
                    f = fb[r, pl.ds((_jg * GW + m) * LANES, LANES)]
                    diff = f - carry[m]
                    a0 = a0 + diff * diff
                    pd = pd + f * carry[GW + m]
                if _jg == 0:
                    a0p_v[rr, :] = a0
                    pdp_v[rr, :] = pd
                else:
                    a0p_v[rr, :] = a0p_v[rr, :] + a0
                    pdp_v[rr, :] = pdp_v[rr, :] + pd
                return carry

            lax.fori_loop(0, CHUNK, row_body, cvals)

    pltpu.async_copy(feat_hbm.at[pl.ds(base, CHUNK), :], fb0, sem0)

    def pair_body(k, _):
        g0 = 2 * k
        g1 = 2 * k + 1
        pltpu.async_copy(
            feat_hbm.at[pl.ds(base + g1 * CHUNK, CHUNK), :], fb1, sem1)
        pltpu.make_async_copy(
            feat_hbm.at[pl.ds(0, CHUNK), :], fb0, sem0).wait()
        process(g0, fb0)
        nxt = jnp.minimum(g0 + 2, NCHUNK - 1)
        pltpu.async_copy(
            feat_hbm.at[pl.ds(base + nxt * CHUNK, CHUNK), :], fb0, sem0)
        pltpu.make_async_copy(
            feat_hbm.at[pl.ds(0, CHUNK), :], fb1, sem1).wait()
        process(g1, fb1)
        return 0

    lax.fori_loop(0, NCHUNK // 2, pair_body, 0)
    pltpu.make_async_copy(feat_hbm.at[pl.ds(0, CHUNK), :], fb0, sem0).wait()

    pltpu.sync_copy(a0p_v, a0p_hbm.at[pl.ds(base, RS), :])
    pltpu.sync_copy(pdp_v, pdp_hbm.at[pl.ds(base, RS), :])


def _sc_partials(feat, cen_flat):
    mesh = plsc.VectorSubcoreMesh(core_axis_name="c", subcore_axis_name="s")
    return pl.kernel(
        _sc_body,
        mesh=mesh,
        out_type=[
            jax.ShapeDtypeStruct((BS, LANES), jnp.float32),
            jax.ShapeDtypeStruct((BS, LANES), jnp.float32),
        ],
        scratch_types=[
            pltpu.VMEM((CLASS_NUM * D,), jnp.float32),
            pltpu.VMEM((D,), jnp.float32),
            pltpu.VMEM((CHUNK, D), jnp.float32),
            pltpu.VMEM((CHUNK, D), jnp.float32),
            pltpu.VMEM((RS, LANES), jnp.float32),
            pltpu.VMEM((RS, LANES), jnp.float32),
            pltpu.SemaphoreType.DMA,
            pltpu.SemaphoreType.DMA,
        ],
    )(feat, cen_flat)


TBLK = 2048
TNBLK = BT // TBLK
SOFF = BS // TBLK


def _tc_main_body(tag_ref, feat_ref, center_ref, out_ref):
    i = pl.program_id(0)
    t = tag_ref[0, 0, :]
    f = feat_ref[...]
    c0 = center_ref[0, :]
    c1 = center_ref[1, :]
    sel = (t[:, None] == 0)
    c = jnp.where(sel, c0[None, :], c1[None, :])
    diff = f - c
    q = jnp.sum(diff * diff, axis=1)
    d = jnp.sqrt(q)
    tf = t.astype(jnp.float32)
    s1 = jnp.sum(d * tf)
    s_all = jnp.sum(d)
    n1 = jnp.sum(tf)

    @pl.when(i == 0)
    def _():
        out_ref[0] = 0.0
        out_ref[1] = 0.0
        out_ref[2] = 0.0

    out_ref[0] += s_all - s1
    out_ref[1] += s1
    out_ref[2] += n1


def _tc_main(tag, feature, center):
    tag3 = tag.reshape(B // TBLK, 1, TBLK)
    return pl.pallas_call(
        _tc_main_body,
        grid=(TNBLK,),
        in_specs=[
            pl.BlockSpec((1, 1, TBLK), lambda i: (i + SOFF, 0, 0)),
            pl.BlockSpec((TBLK, D), lambda i: (i + SOFF, 0)),
            pl.BlockSpec((CLASS_NUM, D), lambda i: (0, 0)),
        ],
        out_specs=pl.BlockSpec(memory_space=pltpu.MemorySpace.SMEM),
        out_shape=jax.ShapeDtypeStruct((3,), jnp.float32),
    )(tag3, feature, center)


FBLK = 2048
FNBLK = BS // FBLK


def _tc_finish_body(tag_ref, a0p_ref, pdp_ref, cen_ref, tcp_ref, out_ref,
                    acc_ref):
    i = pl.program_id(0)
    t = tag_ref[0, 0, :].astype(jnp.float32)
    a0 = jnp.sum(a0p_ref[...], axis=1)
    pd = jnp.sum(pdp_ref[...], axis=1)
    cs = jnp.sum(cen_ref[...] * cen_ref[...], axis=1)
    dcs = cs[1] - cs[0]
    d2 = a0 + t * (2.0 * pd + dcs)
    d = jnp.sqrt(jnp.maximum(d2, 0.0))
    s1 = jnp.sum(d * t)
    s_all = jnp.sum(d)
    n1 = jnp.sum(t)

    @pl.when(i == 0)
    def _():
        acc_ref[0] = 0.0
        acc_ref[1] = 0.0
        acc_ref[2] = 0.0

    acc_ref[0] += s_all - s1
    acc_ref[1] += s1
    acc_ref[2] += n1

    @pl.when(i == FNBLK - 1)
    def _():
        s0_t = acc_ref[0] + tcp_ref[0]
        s1_t = acc_ref[1] + tcp_ref[1]
        n1_t = acc_ref[2] + tcp_ref[2]
        n0_t = jnp.float32(B) - n1_t
        l0 = jnp.where(n0_t > 0, s0_t / jnp.maximum(n0_t, 1.0), 0.0)
        l1 = jnp.where(n1_t > 0, s1_t / jnp.maximum(n1_t, 1.0), 0.0)
        out_ref[0] = l0 + l1


def kernel(tag, feature, center):
    a0p, pdp = _sc_partials(feature, center.reshape(-1))
    tcp = _tc_main(tag, feature, center)
    tag3s = tag[:BS].reshape(FNBLK, 1, FBLK)
    out = pl.pallas_call(
        _tc_finish_body,
        grid=(FNBLK,),
        in_specs=[
            pl.BlockSpec((1, 1, FBLK), lambda i: (i, 0, 0)),
            pl.BlockSpec((FBLK, LANES), lambda i: (i, 0)),
            pl.BlockSpec((FBLK, LANES), lambda i: (i, 0)),
            pl.BlockSpec((CLASS_NUM, D), lambda i: (0, 0)),
            pl.BlockSpec(memory_space=pltpu.MemorySpace.SMEM),
        ],
        out_specs=pl.BlockSpec(memory_space=pltpu.MemorySpace.SMEM),
        out_shape=jax.ShapeDtypeStruct((1,), jnp.float32),
        scratch_shapes=[pltpu.SMEM((3,), jnp.float32)],
    )(tag3s, a0p, pdp, center, tcp)
    return out[0]


# speedup vs baseline: 1.8945x; 1.8945x over previous
import jax
import jax.numpy as jnp
from jax.experimental import pallas as pl
from jax.experimental.pallas import tpu as pltpu

B = 16384
CLASS_NUM = 2
FEATURE_DIM = 1024
BLK = 4096
NBLK = B // BLK


def _body(tag_ref, feat_ref, center_ref, out_ref, acc_ref):
    i = pl.program_id(0)
    t = tag_ref[0, 0, :]
    f = feat_ref[...]
    c0 = center_ref[0, :]
    c1 = center_ref[1, :]
    sel = (t[:, None] == 0)
    c = jnp.where(sel, c0[None, :], c1[None, :])
    diff = f - c
    q = jnp.sum(diff * diff, axis=1)
    d = jnp.sqrt(q)
    tf = t.astype(jnp.float32)
    s1 = jnp.sum(d * tf)
    s_all = jnp.sum(d)
    n1 = jnp.sum(tf)

    @pl.when(i == 0)
    def _():
        acc_ref[0] = 0.0
        acc_ref[1] = 0.0
        acc_ref[2] = 0.0

    acc_ref[0] += s_all - s1
    acc_ref[1] += s1
    acc_ref[2] += n1

    @pl.when(i == NBLK - 1)
    def _():
        s0_t = acc_ref[0]
        s1_t = acc_ref[1]
        n1_t = acc_ref[2]
        n0_t = jnp.float32(B) - n1_t
        l0 = jnp.where(n0_t > 0, s0_t / jnp.maximum(n0_t, 1.0), 0.0)
        l1 = jnp.where(n1_t > 0, s1_t / jnp.maximum(n1_t, 1.0), 0.0)
        out_ref[0] = l0 + l1


def kernel(tag, feature, center):
    tag3 = tag.reshape(NBLK, 1, BLK)
    out = pl.pallas_call(
        _body,
        grid=(NBLK,),
        in_specs=[
            pl.BlockSpec((1, 1, BLK), lambda i: (i, 0, 0)),
            pl.BlockSpec((BLK, FEATURE_DIM), lambda i: (i, 0)),
            pl.BlockSpec((CLASS_NUM, FEATURE_DIM), lambda i: (0, 0)),
        ],
        out_specs=pl.BlockSpec(memory_space=pltpu.MemorySpace.SMEM),
        out_shape=jax.ShapeDtypeStruct((1,), jnp.float32),
        scratch_shapes=[pltpu.SMEM((3,), jnp.float32)],
    )(tag3, feature, center)
    return out[0]


# speedup vs baseline: 1.9479x; 1.0282x over previous
import jax
import jax.numpy as jnp
from jax.experimental import pallas as pl
from jax.experimental.pallas import tpu as pltpu

B = 16384
CLASS_NUM = 2
FEATURE_DIM = 1024
BLK = 2048
NBLK = B // BLK
NSTEP = NBLK // 2


def _part(t, f, c0, c1):
    sel = (t[:, None] == 0)
    c = jnp.where(sel, c0[None, :], c1[None, :])
    diff = f - c
    q = jnp.sum(diff * diff, axis=1)
    d = jnp.sqrt(q)
    tf = t.astype(jnp.float32)
    s1 = jnp.sum(d * tf)
    return jnp.sum(d) - s1, s1, jnp.sum(tf)


def _body(taga_ref, tagb_ref, feata_ref, featb_ref, center_ref, out_ref,
          acc_ref):
    i = pl.program_id(0)
    c0 = center_ref[0, :]
    c1 = center_ref[1, :]
    s0a, s1a, n1a = _part(taga_ref[0, 0, :], feata_ref[...], c0, c1)
    s0b, s1b, n1b = _part(tagb_ref[0, 0, :], featb_ref[...], c0, c1)

    @pl.when(i == 0)
    def _():
        acc_ref[0] = 0.0
        acc_ref[1] = 0.0
        acc_ref[2] = 0.0

    acc_ref[0] += s0a + s0b
    acc_ref[1] += s1a + s1b
    acc_ref[2] += n1a + n1b

    @pl.when(i == NSTEP - 1)
    def _():
        s0_t = acc_ref[0]
        s1_t = acc_ref[1]
        n1_t = acc_ref[2]
        n0_t = jnp.float32(B) - n1_t
        l0 = jnp.where(n0_t > 0, s0_t / jnp.maximum(n0_t, 1.0), 0.0)
        l1 = jnp.where(n1_t > 0, s1_t / jnp.maximum(n1_t, 1.0), 0.0)
        out_ref[0] = l0 + l1


def kernel(tag, feature, center):
    tag3 = tag.reshape(NBLK, 1, BLK)
    out = pl.pallas_call(
        _body,
        grid=(NSTEP,),
        in_specs=[
            pl.BlockSpec((1, 1, BLK), lambda i: (2 * i, 0, 0)),
            pl.BlockSpec((1, 1, BLK), lambda i: (2 * i + 1, 0, 0)),
            pl.BlockSpec((BLK, FEATURE_DIM), lambda i: (2 * i, 0)),
            pl.BlockSpec((BLK, FEATURE_DIM), lambda i: (2 * i + 1, 0)),
            pl.BlockSpec((CLASS_NUM, FEATURE_DIM), lambda i: (0, 0)),
        ],
        out_specs=pl.BlockSpec(memory_space=pltpu.MemorySpace.SMEM),
        out_shape=jax.ShapeDtypeStruct((1,), jnp.float32),
        scratch_shapes=[pltpu.SMEM((3,), jnp.float32)],
    )(tag3, tag3, feature, feature, center)
    return out[0]
